# baseline (speedup 1.0000x reference)
import functools

import jax
import jax.numpy as jnp
from jax import lax
from jax.experimental import pallas as pl
from jax.experimental.pallas import tpu as pltpu
from jax.experimental.pallas import tpu_sc as plsc

N = 10000
E = 320000
D = 128

NC = 2
NS = 16
NW = NC * NS
EPW = E // NW
C = 96
NCHUNK = 108
EPWP = NCHUNK * C
NP = N + 8
OWN = 1000
ZR = 250

_mesh = plsc.VectorSubcoreMesh(
    core_axis_name="c", subcore_axis_name="s", num_cores=NC, num_subcores=NS)
_sc_params = pltpu.CompilerParams(needs_layout_passes=False,
                                  use_tc_tiling_on_sc=False)


@functools.partial(
    pl.kernel,
    out_type=jax.ShapeDtypeStruct((NC * N, 16), jnp.float32),
    mesh=_mesh,
    scratch_types=[
        pltpu.VMEM((2, C), jnp.int32),
        pltpu.VMEM((2, C), jnp.int32),
        pltpu.VMEM((2, C), jnp.int32),
        pltpu.VMEM((C, 16), jnp.float32),
        pltpu.VMEM((ZR, 16), jnp.float32),
        pltpu.SemaphoreType.DMA,
        pltpu.SemaphoreType.DMA,
        pltpu.SemaphoreType.DMA,
        pltpu.SemaphoreType.DMA,
        pltpu.SemaphoreType.DMA,
        pltpu.SemaphoreType.DMA,
        pltpu.SemaphoreType.DMA,
        pltpu.SemaphoreType.DMA,
        pltpu.SemaphoreType.DMA,
        pltpu.VMEM_SHARED((NP, 16), jnp.float32),
    ],
    compiler_params=_sc_params,
)
def _sc_degree(dst_hbm, deg_out, dxA, dxB, dxC, ones_v, zbuf_v,
               ipsA, ipsB, ipsC, s0, s1, s2, s3, s4, s5, acc_sh):
    cid = lax.axis_index("c")
    sid = lax.axis_index("s")
    wid = sid * NC + cid

    def ones_body(r, _):
        ones_v[r, :] = jnp.ones((16,), jnp.float32)
        return 0
    lax.fori_loop(0, C, ones_body, 0)

    def zfill_body(r, _):
        zbuf_v[r, :] = jnp.zeros((16,), jnp.float32)
        return 0
    lax.fori_loop(0, ZR, zfill_body, 0)

    @pl.when(sid < N // OWN)
    def _():
        def zero_body(k, _):
            pltpu.sync_copy(zbuf_v,
                            acc_sh.at[pl.ds(sid * OWN + k * ZR, ZR)])
            return 0
        lax.fori_loop(0, OWN // ZR, zero_body, 0)
    plsc.subcore_barrier()

    def issue_pair(p, dx, sem):
        row = wid * NCHUNK + 2 * p
        pltpu.async_copy(dst_hbm.at[pl.ds(row, 2)], dx, sem)

    def wait_pair(dx, sem):
        pltpu.make_async_copy(dst_hbm.at[pl.ds(0, 2)], dx, sem).wait()

    def issue_scat(dx, j, sem):
        pltpu.async_copy(ones_v, acc_sh.at[dx.at[j]], sem, add=True)

    def wait_scat(dx, j, sem):
        pltpu.make_async_copy(ones_v, acc_sh.at[dx.at[j]], sem).wait()

    issue_pair(0, dxA, ipsA)
    wait_pair(dxA, ipsA)
    issue_pair(1, dxB, ipsB)

    def pipe_body(k, _):
        issue_scat(dxA, 0, s0)
        issue_scat(dxA, 1, s1)
        wait_pair(dxB, ipsB)
        issue_scat(dxB, 0, s2)
        issue_scat(dxB, 1, s3)
        issue_pair(3 * k + 2, dxC, ipsC)
        wait_scat(dxA, 0, s0)
        wait_scat(dxA, 1, s1)
        issue_pair(3 * k + 3, dxA, ipsA)
        wait_pair(dxC, ipsC)
        issue_scat(dxC, 0, s4)
        issue_scat(dxC, 1, s5)
        wait_scat(dxB, 0, s2)
        wait_scat(dxB, 1, s3)
        issue_pair(3 * k + 4, dxB, ipsB)
        wait_scat(dxC, 0, s4)
        wait_scat(dxC, 1, s5)
        wait_pair(dxA, ipsA)
        return 0
    lax.fori_loop(0, NCHUNK // 6, pipe_body, 0)

    wait_pair(dxB, ipsB)
    plsc.subcore_barrier()

    @pl.when(sid < N // OWN)
    def _():
        pltpu.sync_copy(acc_sh.at[pl.ds(sid * OWN, OWN)],
                        deg_out.at[pl.ds(cid * N + sid * OWN, OWN)])


def _tc_scalars_body(nf_ref, hid_ref, sw_ref, q_ref, degp_ref,
                     hu_ref, hv_ref, nd_ref, hp_ref):
    deg = degp_ref[0:N, 0:1] + degp_ref[N:2 * N, 0:1]
    nd = lax.rsqrt(deg + 1.0)
    nd_ref[...] = nd
    c = nd / (q_ref[...] * float(E))
    zrow = jnp.zeros((NP - N, D), jnp.float32)
    hp_ref[...] = jnp.concatenate([hid_ref[...] * c, zrow], axis=0)
    nf = nf_ref[...]
    hu_ref[...] = jnp.dot(nf, sw_ref[:, 0:1],
                          preferred_element_type=jnp.float32)
    hv_ref[...] = jnp.dot(nf, sw_ref[:, 1:2],
                          preferred_element_type=jnp.float32)


def _tc_scalars(node_feat, hidden_feat, sample_weights, q2, deg_part):
    return pl.pallas_call(
        _tc_scalars_body,
        out_shape=(jax.ShapeDtypeStruct((N, 1), jnp.float32),
                   jax.ShapeDtypeStruct((N, 1), jnp.float32),
                   jax.ShapeDtypeStruct((N, 1), jnp.float32),
                   jax.ShapeDtypeStruct((NP, D), jnp.float32)),
    )(node_feat, hidden_feat, sample_weights, q2, deg_part)


@functools.partial(
    pl.kernel,
    out_type=jax.ShapeDtypeStruct((NC * N, D), jnp.float32),
    mesh=_mesh,
    scratch_types=[
        pltpu.VMEM((2, C), jnp.int32),
        pltpu.VMEM((2, C), jnp.int32),
        pltpu.VMEM((2, C), jnp.int32),
        pltpu.VMEM((2, C), jnp.int32),
        pltpu.VMEM((C, D), jnp.float32),
        pltpu.VMEM((C, D), jnp.float32),
        pltpu.VMEM((N + 16,), jnp.float32),
        pltpu.VMEM((N + 16,), jnp.float32),
        pltpu.SemaphoreType.DMA,
        pltpu.SemaphoreType.DMA,
        pltpu.SemaphoreType.DMA,
        pltpu.SemaphoreType.DMA,
        pltpu.VMEM_SHARED((NP, D), jnp.float32),
    ],
    compiler_params=_sc_params,
)
def _sc_edges(hp_hbm, src_hbm, dst_hbm, hu_hbm, hv_hbm, zeros_hbm, part_out,
              sxA, dxA, sxB, dxB, rows0, rows1, hu_v, hv_v,
              ipsA, ipsB, gs0, gs1, acc_sh):
    cid = lax.axis_index("c")
    sid = lax.axis_index("s")
    wid = sid * NC + cid

    pltpu.sync_copy(hu_hbm, hu_v)
    pltpu.sync_copy(hv_hbm, hv_v)

    @pl.when(sid < N // OWN)
    def _():
        def zero_body(k, _):
            pltpu.sync_copy(zeros_hbm,
                            acc_sh.at[pl.ds(sid * OWN + k * ZR, ZR)])
            return 0
        lax.fori_loop(0, OWN // ZR, zero_body, 0)
    plsc.subcore_barrier()

    def issue_pair(p, sx, dx, sem):
        row = wid * NCHUNK + 2 * p
        pltpu.async_copy(src_hbm.at[pl.ds(row, 2)], sx, sem)
        pltpu.async_copy(dst_hbm.at[pl.ds(row, 2)], dx, sem)

    def wait_pair(sx, dx, sem):
        pltpu.make_async_copy(src_hbm.at[pl.ds(0, 2)], sx, sem).wait()
        pltpu.make_async_copy(dst_hbm.at[pl.ds(0, 2)], dx, sem).wait()

    def issue_gather(sx, j, rows, sem):
        pltpu.async_copy(hp_hbm.at[sx.at[j]], rows, sem)

    def wait_gather(sx, j, rows, sem):
        pltpu.make_async_copy(hp_hbm.at[sx.at[j]], rows, sem).wait()

    def compute_scatter(sx, dx, j, rows):
        def grp_body(g, _):
            s16 = pl.ds(g * 16, 16)
            si = sx[j, s16]
            di = dx[j, s16]
            hu16 = plsc.load_gather(hu_v, [si])
            hv16 = plsc.load_gather(hv_v, [di])
            s = jnp.maximum(hu16 + hv16, 0.0) + 0.1
            for e in range(16):
                a = s[e]
                row = g * 16 + e
                for jj in range(D // 16):
                    sl = pl.ds(jj * 16, 16)
                    rows[row, sl] = rows[row, sl] * a
            return 0
        lax.fori_loop(0, C // 16, grp_body, 0)
        pltpu.sync_copy(rows, acc_sh.at[dx.at[j]], add=True)

    NPAIR_LOOP = NCHUNK // 4

    issue_pair(0, sxA, dxA, ipsA)
    wait_pair(sxA, dxA, ipsA)
    issue_gather(sxA, 0, rows0, gs0)
    issue_pair(1, sxB, dxB, ipsB)

    def pipe_body(k, _):
        issue_gather(sxA, 1, rows1, gs1)
        wait_gather(sxA, 0, rows0, gs0)
        compute_scatter(sxA, dxA, 0, rows0)
        wait_pair(sxB, dxB, ipsB)
        issue_gather(sxB, 0, rows0, gs0)
        wait_gather(sxA, 1, rows1, gs1)
        compute_scatter(sxA, dxA, 1, rows1)
        issue_pair(2 * k + 2, sxA, dxA, ipsA)
        issue_gather(sxB, 1, rows1, gs1)
        wait_gather(sxB, 0, rows0, gs0)
        compute_scatter(sxB, dxB, 0, rows0)
        wait_pair(sxA, dxA, ipsA)
        issue_gather(sxA, 0, rows0, gs0)
        wait_gather(sxB, 1, rows1, gs1)
        compute_scatter(sxB, dxB, 1, rows1)
        issue_pair(2 * k + 3, sxB, dxB, ipsB)
        return 0
    lax.fori_loop(0, NPAIR_LOOP, pipe_body, 0)

    wait_gather(sxA, 0, rows0, gs0)
    wait_pair(sxB, dxB, ipsB)

    plsc.subcore_barrier()

    @pl.when(sid < N // OWN)
    def _():
        pltpu.sync_copy(acc_sh.at[pl.ds(sid * OWN, OWN)],
                        part_out.at[pl.ds(cid * N + sid * OWN, OWN)])


def _tc_final_body(part_ref, nd_ref, w_ref, b_ref, out_ref):
    neigh = (part_ref[0:N, :] + part_ref[N:2 * N, :]) * nd_ref[...]
    out_ref[...] = lax.dot_general(
        neigh, w_ref[...], (((1,), (1,)), ((), ())),
        preferred_element_type=jnp.float32) + b_ref[...]


def _tc_final(part, nd2, W, b2):
    return pl.pallas_call(
        _tc_final_body,
        out_shape=jax.ShapeDtypeStruct((N, D), jnp.float32),
    )(part, nd2, W, b2)


def kernel(hidden_feat, node_feat, edge_index, sample_weights, q_probs, W, b):
    src = edge_index[0].astype(jnp.int32).reshape(NW, EPW)
    dst = edge_index[1].astype(jnp.int32).reshape(NW, EPW)
    dpad = jnp.full((NW, EPWP - EPW), N, jnp.int32)
    tail = jnp.full((6, C), N, jnp.int32)
    src2d = jnp.concatenate(
        [jnp.concatenate([src, dpad], axis=1).reshape(NW * NCHUNK, C),
         tail], axis=0)
    dst2d = jnp.concatenate(
        [jnp.concatenate([dst, dpad], axis=1).reshape(NW * NCHUNK, C),
         tail], axis=0)

    deg_part = _sc_degree(dst2d)
    hu, hv, nd, hprime = _tc_scalars(
        node_feat, hidden_feat, sample_weights,
        q_probs.reshape(N, 1), deg_part)
    z16 = jnp.zeros(16, jnp.float32)
    hu_pad = jnp.concatenate([hu.reshape(N), z16])
    hv_pad = jnp.concatenate([hv.reshape(N), z16])
    zeros = jnp.zeros((ZR, D), jnp.float32)
    part = _sc_edges(hprime, src2d, dst2d, hu_pad, hv_pad, zeros)
    return _tc_final(part, nd, W, b.reshape(1, D))


# speedup vs baseline: 1.0023x; 1.0023x over previous
import functools

import jax
import jax.numpy as jnp
from jax import lax
from jax.experimental import pallas as pl
from jax.experimental.pallas import tpu as pltpu
from jax.experimental.pallas import tpu_sc as plsc

N = 10000
E = 320000
D = 128

NC = 2
NS = 16
NW = NC * NS
EPW = E // NW
C = 96
NCHUNK = 108
EPWP = NCHUNK * C
NP = N + 8
CD = 80
NCHD = 126
OWN = 1000
ZR = 250

_mesh = plsc.VectorSubcoreMesh(
    core_axis_name="c", subcore_axis_name="s", num_cores=NC, num_subcores=NS)
_sc_params = pltpu.CompilerParams(needs_layout_passes=False,
                                  use_tc_tiling_on_sc=False)


@functools.partial(
    pl.kernel,
    out_type=jax.ShapeDtypeStruct((NC * N, 16), jnp.float32),
    mesh=_mesh,
    scratch_types=[
        pltpu.VMEM((2, CD), jnp.int32),
        pltpu.VMEM((2, CD), jnp.int32),
        pltpu.VMEM((2, CD), jnp.int32),
        pltpu.VMEM((CD, 16), jnp.float32),
        pltpu.VMEM((ZR, 16), jnp.float32),
        pltpu.SemaphoreType.DMA,
        pltpu.SemaphoreType.DMA,
        pltpu.SemaphoreType.DMA,
        pltpu.SemaphoreType.DMA,
        pltpu.SemaphoreType.DMA,
        pltpu.SemaphoreType.DMA,
        pltpu.SemaphoreType.DMA,
        pltpu.SemaphoreType.DMA,
        pltpu.SemaphoreType.DMA,
        pltpu.VMEM_SHARED((NP, 16), jnp.float32),
    ],
    compiler_params=_sc_params,
)
def _sc_degree(dst_hbm, deg_out, dxA, dxB, dxC, ones_v, zbuf_v,
               ipsA, ipsB, ipsC, s0, s1, s2, s3, s4, s5, acc_sh):
    cid = lax.axis_index("c")
    sid = lax.axis_index("s")
    wid = sid * NC + cid

    def ones_body(r, _):
        ones_v[r, :] = jnp.ones((16,), jnp.float32)
        return 0
    lax.fori_loop(0, CD, ones_body, 0)

    def zfill_body(r, _):
        zbuf_v[r, :] = jnp.zeros((16,), jnp.float32)
        return 0
    lax.fori_loop(0, ZR, zfill_body, 0)

    @pl.when(sid < N // OWN)
    def _():
        def zero_body(k, _):
            pltpu.sync_copy(zbuf_v,
                            acc_sh.at[pl.ds(sid * OWN + k * ZR, ZR)])
            return 0
        lax.fori_loop(0, OWN // ZR, zero_body, 0)
    plsc.subcore_barrier()

    def issue_pair(p, dx, sem):
        row = wid * NCHD + 2 * p
        pltpu.async_copy(dst_hbm.at[pl.ds(row, 2)], dx, sem)

    def wait_pair(dx, sem):
        pltpu.make_async_copy(dst_hbm.at[pl.ds(0, 2)], dx, sem).wait()

    def issue_scat(dx, j, sem):
        pltpu.async_copy(ones_v, acc_sh.at[dx.at[j]], sem, add=True)

    def wait_scat(dx, j, sem):
        pltpu.make_async_copy(ones_v, acc_sh.at[dx.at[j]], sem).wait()

    issue_pair(0, dxA, ipsA)
    wait_pair(dxA, ipsA)
    issue_pair(1, dxB, ipsB)

    def pipe_body(k, _):
        issue_scat(dxA, 0, s0)
        issue_scat(dxA, 1, s1)
        wait_pair(dxB, ipsB)
        issue_scat(dxB, 0, s2)
        issue_scat(dxB, 1, s3)
        issue_pair(3 * k + 2, dxC, ipsC)
        wait_scat(dxA, 0, s0)
        wait_scat(dxA, 1, s1)
        issue_pair(3 * k + 3, dxA, ipsA)
        wait_pair(dxC, ipsC)
        issue_scat(dxC, 0, s4)
        issue_scat(dxC, 1, s5)
        wait_scat(dxB, 0, s2)
        wait_scat(dxB, 1, s3)
        issue_pair(3 * k + 4, dxB, ipsB)
        wait_scat(dxC, 0, s4)
        wait_scat(dxC, 1, s5)
        wait_pair(dxA, ipsA)
        return 0
    lax.fori_loop(0, NCHD // 6, pipe_body, 0)

    wait_pair(dxB, ipsB)
    plsc.subcore_barrier()

    @pl.when(sid < N // OWN)
    def _():
        pltpu.sync_copy(acc_sh.at[pl.ds(sid * OWN, OWN)],
                        deg_out.at[pl.ds(cid * N + sid * OWN, OWN)])


def _tc_scalars_body(nf_ref, hid_ref, sw_ref, q_ref, degp_ref,
                     hu_ref, hv_ref, nd_ref, hp_ref):
    deg = degp_ref[0:N, 0:1] + degp_ref[N:2 * N, 0:1]
    nd = lax.rsqrt(deg + 1.0)
    nd_ref[...] = nd
    c = nd / (q_ref[...] * float(E))
    zrow = jnp.zeros((NP - N, D), jnp.float32)
    hp_ref[...] = jnp.concatenate([hid_ref[...] * c, zrow], axis=0)
    nf = nf_ref[...]
    hu_ref[...] = jnp.dot(nf, sw_ref[:, 0:1],
                          preferred_element_type=jnp.float32)
    hv_ref[...] = jnp.dot(nf, sw_ref[:, 1:2],
                          preferred_element_type=jnp.float32)


def _tc_scalars(node_feat, hidden_feat, sample_weights, q2, deg_part):
    return pl.pallas_call(
        _tc_scalars_body,
        out_shape=(jax.ShapeDtypeStruct((N, 1), jnp.float32),
                   jax.ShapeDtypeStruct((N, 1), jnp.float32),
                   jax.ShapeDtypeStruct((N, 1), jnp.float32),
                   jax.ShapeDtypeStruct((NP, D), jnp.float32)),
    )(node_feat, hidden_feat, sample_weights, q2, deg_part)


@functools.partial(
    pl.kernel,
    out_type=jax.ShapeDtypeStruct((NC * N, D), jnp.float32),
    mesh=_mesh,
    scratch_types=[
        pltpu.VMEM((2, C), jnp.int32),
        pltpu.VMEM((2, C), jnp.int32),
        pltpu.VMEM((2, C), jnp.int32),
        pltpu.VMEM((2, C), jnp.int32),
        pltpu.VMEM((C, D), jnp.float32),
        pltpu.VMEM((C, D), jnp.float32),
        pltpu.VMEM((N + 16,), jnp.float32),
        pltpu.VMEM((N + 16,), jnp.float32),
        pltpu.SemaphoreType.DMA,
        pltpu.SemaphoreType.DMA,
        pltpu.SemaphoreType.DMA,
        pltpu.SemaphoreType.DMA,
        pltpu.VMEM_SHARED((N, D), jnp.float32),
    ],
    compiler_params=_sc_params,
)
def _sc_edges(hp_hbm, src_hbm, dst_hbm, hu_hbm, hv_hbm, zeros_hbm, part_out,
              sxA, dxA, sxB, dxB, rows0, rows1, hu_v, hv_v,
              ipsA, ipsB, gs0, gs1, acc_sh):
    cid = lax.axis_index("c")
    sid = lax.axis_index("s")
    wid = sid * NC + cid

    pltpu.sync_copy(hu_hbm, hu_v)
    pltpu.sync_copy(hv_hbm, hv_v)

    @pl.when(sid < N // OWN)
    def _():
        def zero_body(k, _):
            pltpu.sync_copy(zeros_hbm,
                            acc_sh.at[pl.ds(sid * OWN + k * ZR, ZR)])
            return 0
        lax.fori_loop(0, OWN // ZR, zero_body, 0)
    plsc.subcore_barrier()

    def issue_pair(p, sx, dx, sem):
        row = wid * NCHUNK + 2 * p
        pltpu.async_copy(src_hbm.at[pl.ds(row, 2)], sx, sem)
        pltpu.async_copy(dst_hbm.at[pl.ds(row, 2)], dx, sem)

    def wait_pair(sx, dx, sem):
        pltpu.make_async_copy(src_hbm.at[pl.ds(0, 2)], sx, sem).wait()
        pltpu.make_async_copy(dst_hbm.at[pl.ds(0, 2)], dx, sem).wait()

    def issue_gather(sx, j, rows, sem):
        pltpu.async_copy(hp_hbm.at[sx.at[j]], rows, sem)

    def wait_gather(sx, j, rows, sem):
        pltpu.make_async_copy(hp_hbm.at[sx.at[j]], rows, sem).wait()

    def compute_scatter(sx, dx, j, rows):
        def grp_body(g, _):
            s16 = pl.ds(g * 16, 16)
            si = sx[j, s16]
            di = dx[j, s16]
            hu16 = plsc.load_gather(hu_v, [si])
            hv16 = plsc.load_gather(hv_v, [di])
            s = jnp.maximum(hu16 + hv16, 0.0) + 0.1
            for e in range(16):
                a = s[e]
                row = g * 16 + e
                for jj in range(D // 16):
                    sl = pl.ds(jj * 16, 16)
                    rows[row, sl] = rows[row, sl] * a
            return 0
        lax.fori_loop(0, C // 16, grp_body, 0)
        pltpu.sync_copy(rows, acc_sh.at[dx.at[j]], add=True)

    NPAIR_LOOP = NCHUNK // 4

    issue_pair(0, sxA, dxA, ipsA)
    wait_pair(sxA, dxA, ipsA)
    issue_gather(sxA, 0, rows0, gs0)
    issue_pair(1, sxB, dxB, ipsB)

    def pipe_body(k, _):
        issue_gather(sxA, 1, rows1, gs1)
        wait_gather(sxA, 0, rows0, gs0)
        compute_scatter(sxA, dxA, 0, rows0)
        wait_pair(sxB, dxB, ipsB)
        issue_gather(sxB, 0, rows0, gs0)
        wait_gather(sxA, 1, rows1, gs1)
        compute_scatter(sxA, dxA, 1, rows1)
        issue_pair(2 * k + 2, sxA, dxA, ipsA)
        issue_gather(sxB, 1, rows1, gs1)
        wait_gather(sxB, 0, rows0, gs0)
        compute_scatter(sxB, dxB, 0, rows0)
        wait_pair(sxA, dxA, ipsA)
        issue_gather(sxA, 0, rows0, gs0)
        wait_gather(sxB, 1, rows1, gs1)
        compute_scatter(sxB, dxB, 1, rows1)
        issue_pair(2 * k + 3, sxB, dxB, ipsB)
        return 0
    lax.fori_loop(0, NPAIR_LOOP, pipe_body, 0)

    wait_gather(sxA, 0, rows0, gs0)
    wait_pair(sxB, dxB, ipsB)

    plsc.subcore_barrier()

    @pl.when(sid < N // OWN)
    def _():
        pltpu.sync_copy(acc_sh.at[pl.ds(sid * OWN, OWN)],
                        part_out.at[pl.ds(cid * N + sid * OWN, OWN)])


def _tc_final_body(part_ref, nd_ref, w_ref, b_ref, out_ref):
    neigh = (part_ref[0:N, :] + part_ref[N:2 * N, :]) * nd_ref[...]
    out_ref[...] = lax.dot_general(
        neigh, w_ref[...], (((1,), (1,)), ((), ())),
        preferred_element_type=jnp.float32) + b_ref[...]


def _tc_final(part, nd2, W, b2):
    return pl.pallas_call(
        _tc_final_body,
        out_shape=jax.ShapeDtypeStruct((N, D), jnp.float32),
    )(part, nd2, W, b2)


def kernel(hidden_feat, node_feat, edge_index, sample_weights, q_probs, W, b):
    src = edge_index[0].astype(jnp.int32).reshape(NW, EPW)
    dst = edge_index[1].astype(jnp.int32).reshape(NW, EPW)
    dpad_deg = jnp.full((NW, NCHD * CD - EPW), N, jnp.int32)
    tail_deg = jnp.full((6, CD), N, jnp.int32)
    dst2d_deg = jnp.concatenate(
        [jnp.concatenate([dst, dpad_deg], axis=1).reshape(NW * NCHD, CD),
         tail_deg], axis=0)
    npad = EPWP - EPW
    spad = jnp.full((NW, npad), N, jnp.int32)
    dspread = jnp.broadcast_to((jnp.arange(npad, dtype=jnp.int32) * 97) % N,
                               (NW, npad))
    tail = jnp.zeros((6, C), jnp.int32)
    src2d = jnp.concatenate(
        [jnp.concatenate([src, spad], axis=1).reshape(NW * NCHUNK, C),
         jnp.full((6, C), N, jnp.int32)], axis=0)
    dst2d = jnp.concatenate(
        [jnp.concatenate([dst, dspread], axis=1).reshape(NW * NCHUNK, C),
         tail], axis=0)

    deg_part = _sc_degree(dst2d_deg)
    hu, hv, nd, hprime = _tc_scalars(
        node_feat, hidden_feat, sample_weights,
        q_probs.reshape(N, 1), deg_part)
    z16 = jnp.zeros(16, jnp.float32)
    hu_pad = jnp.concatenate([hu.reshape(N), z16])
    hv_pad = jnp.concatenate([hv.reshape(N), z16])
    zeros = jnp.zeros((ZR, D), jnp.float32)
    part = _sc_edges(hprime, src2d, dst2d, hu_pad, hv_pad, zeros)
    return _tc_final(part, nd, W, b.reshape(1, D))


# speedup vs baseline: 2.8721x; 2.8656x over previous
import functools

import jax
import jax.numpy as jnp
from jax import lax
from jax.experimental import pallas as pl
from jax.experimental.pallas import tpu as pltpu
from jax.experimental.pallas import tpu_sc as plsc

N = 10000
E = 320000
D = 128

NC = 2
NS = 16
NW = NC * NS
EPW = E // NW
C = 80
NCHUNK = EPW // C + 1
NP = N + 8
OWN = 1000
ZR = 250

_mesh = plsc.VectorSubcoreMesh(
    core_axis_name="c", subcore_axis_name="s", num_cores=NC, num_subcores=NS)
_sc_params = pltpu.CompilerParams(needs_layout_passes=False,
                                  use_tc_tiling_on_sc=False)


@functools.partial(
    pl.kernel,
    out_type=jax.ShapeDtypeStruct((NC * N, 16), jnp.float32),
    mesh=_mesh,
    scratch_types=[
        pltpu.VMEM((2, C), jnp.int32),
        pltpu.VMEM((2, C), jnp.int32),
        pltpu.VMEM((2, C), jnp.int32),
        pltpu.VMEM((C, 16), jnp.float32),
        pltpu.VMEM((ZR, 16), jnp.float32),
        pltpu.SemaphoreType.DMA,
        pltpu.SemaphoreType.DMA,
        pltpu.SemaphoreType.DMA,
        pltpu.SemaphoreType.DMA,
        pltpu.SemaphoreType.DMA,
        pltpu.SemaphoreType.DMA,
        pltpu.SemaphoreType.DMA,
        pltpu.SemaphoreType.DMA,
        pltpu.SemaphoreType.DMA,
        pltpu.VMEM_SHARED((NP, 16), jnp.float32),
    ],
    compiler_params=_sc_params,
)
def _sc_degree(dst_hbm, deg_out, dxA, dxB, dxC, ones_v, zbuf_v,
               ipsA, ipsB, ipsC, s0, s1, s2, s3, s4, s5, acc_sh):
    cid = lax.axis_index("c")
    sid = lax.axis_index("s")
    wid = sid * NC + cid

    def ones_body(r, _):
        ones_v[r, :] = jnp.ones((16,), jnp.float32)
        return 0
    lax.fori_loop(0, C, ones_body, 0)

    def zfill_body(r, _):
        zbuf_v[r, :] = jnp.zeros((16,), jnp.float32)
        return 0
    lax.fori_loop(0, ZR, zfill_body, 0)

    @pl.when(sid < N // OWN)
    def _():
        def zero_body(k, _):
            pltpu.sync_copy(zbuf_v,
                            acc_sh.at[pl.ds(sid * OWN + k * ZR, ZR)])
            return 0
        lax.fori_loop(0, OWN // ZR, zero_body, 0)
    plsc.subcore_barrier()

    def issue_pair(p, dx, sem):
        row = wid * NCHUNK + 2 * p
        pltpu.async_copy(dst_hbm.at[pl.ds(row, 2)], dx, sem)

    def wait_pair(dx, sem):
        pltpu.make_async_copy(dst_hbm.at[pl.ds(0, 2)], dx, sem).wait()

    def issue_scat(dx, j, sem):
        pltpu.async_copy(ones_v, acc_sh.at[dx.at[j]], sem, add=True)

    def wait_scat(dx, j, sem):
        pltpu.make_async_copy(ones_v, acc_sh.at[dx.at[j]], sem).wait()

    issue_pair(0, dxA, ipsA)
    wait_pair(dxA, ipsA)
    issue_pair(1, dxB, ipsB)

    def pipe_body(k, _):
        issue_scat(dxA, 0, s0)
        issue_scat(dxA, 1, s1)
        wait_pair(dxB, ipsB)
        issue_scat(dxB, 0, s2)
        issue_scat(dxB, 1, s3)
        issue_pair(3 * k + 2, dxC, ipsC)
        wait_scat(dxA, 0, s0)
        wait_scat(dxA, 1, s1)
        issue_pair(3 * k + 3, dxA, ipsA)
        wait_pair(dxC, ipsC)
        issue_scat(dxC, 0, s4)
        issue_scat(dxC, 1, s5)
        wait_scat(dxB, 0, s2)
        wait_scat(dxB, 1, s3)
        issue_pair(3 * k + 4, dxB, ipsB)
        wait_scat(dxC, 0, s4)
        wait_scat(dxC, 1, s5)
        wait_pair(dxA, ipsA)
        return 0
    lax.fori_loop(0, NCHUNK // 6, pipe_body, 0)

    wait_pair(dxB, ipsB)
    plsc.subcore_barrier()

    @pl.when(sid < N // OWN)
    def _():
        pltpu.sync_copy(acc_sh.at[pl.ds(sid * OWN, OWN)],
                        deg_out.at[pl.ds(cid * N + sid * OWN, OWN)])


def _tc_matvec_body(nf_ref, sw_ref, hu_ref, hv_ref):
    nf = nf_ref[...]
    hu_ref[...] = jnp.dot(nf, sw_ref[:, 0:1],
                          preferred_element_type=jnp.float32)
    hv_ref[...] = jnp.dot(nf, sw_ref[:, 1:2],
                          preferred_element_type=jnp.float32)


def _tc_matvec(node_feat, sample_weights):
    out = jax.ShapeDtypeStruct((N, 1), jnp.float32)
    return pl.pallas_call(
        _tc_matvec_body,
        out_shape=(out, out),
    )(node_feat, sample_weights)


def _tc_scalars_body(hid_ref, q_ref, degp_ref, nd_ref, hp_ref):
    deg = degp_ref[0:N, 0:1] + degp_ref[N:2 * N, 0:1]
    nd = lax.rsqrt(deg + 1.0)
    nd_ref[...] = nd
    c = nd / (q_ref[...] * float(E))
    zrow = jnp.zeros((NP - N, D), jnp.float32)
    hp_ref[...] = jnp.concatenate([hid_ref[...] * c, zrow], axis=0)


def _tc_scalars(hidden_feat, q2, deg_part):
    return pl.pallas_call(
        _tc_scalars_body,
        out_shape=(jax.ShapeDtypeStruct((N, 1), jnp.float32),
                   jax.ShapeDtypeStruct((NP, D), jnp.float32)),
    )(hidden_feat, q2, deg_part)


@functools.partial(
    pl.kernel,
    out_type=jax.ShapeDtypeStruct((NC * N, D), jnp.float32),
    mesh=_mesh,
    scratch_types=[
        pltpu.VMEM((2, C), jnp.int32),
        pltpu.VMEM((2, C), jnp.int32),
        pltpu.VMEM((2, C), jnp.int32),
        pltpu.VMEM((2, C), jnp.int32),
        pltpu.VMEM((C, D), jnp.float32),
        pltpu.VMEM((C, D), jnp.float32),
        pltpu.VMEM((N + 16,), jnp.float32),
        pltpu.VMEM((N + 16,), jnp.float32),
        pltpu.SemaphoreType.DMA,
        pltpu.SemaphoreType.DMA,
        pltpu.SemaphoreType.DMA,
        pltpu.SemaphoreType.DMA,
        pltpu.VMEM_SHARED((NP, D), jnp.float32),
    ],
    compiler_params=_sc_params,
)
def _sc_edges(hp_hbm, src_hbm, dst_hbm, hu_hbm, hv_hbm, zeros_hbm, part_out,
              sxA, dxA, sxB, dxB, rows0, rows1, hu_v, hv_v,
              ipsA, ipsB, gs0, gs1, acc_sh):
    cid = lax.axis_index("c")
    sid = lax.axis_index("s")
    wid = sid * NC + cid

    pltpu.sync_copy(hu_hbm, hu_v)
    pltpu.sync_copy(hv_hbm, hv_v)

    @pl.when(sid < N // OWN)
    def _():
        def zero_body(k, _):
            pltpu.sync_copy(zeros_hbm,
                            acc_sh.at[pl.ds(sid * OWN + k * ZR, ZR)])
            return 0
        lax.fori_loop(0, OWN // ZR, zero_body, 0)
    plsc.subcore_barrier()

    def issue_pair(p, sx, dx, sem):
        row = wid * NCHUNK + 2 * p
        pltpu.async_copy(src_hbm.at[pl.ds(row, 2)], sx, sem)
        pltpu.async_copy(dst_hbm.at[pl.ds(row, 2)], dx, sem)

    def wait_pair(sx, dx, sem):
        pltpu.make_async_copy(src_hbm.at[pl.ds(0, 2)], sx, sem).wait()
        pltpu.make_async_copy(dst_hbm.at[pl.ds(0, 2)], dx, sem).wait()

    def issue_gather(sx, j, rows, sem):
        pltpu.async_copy(hp_hbm.at[sx.at[j]], rows, sem)

    def wait_gather(sx, j, rows, sem):
        pltpu.make_async_copy(hp_hbm.at[sx.at[j]], rows, sem).wait()

    def compute_scatter(sx, dx, j, rows):
        def grp_body(g, _):
            s16 = pl.ds(g * 16, 16)
            si = sx[j, s16]
            di = dx[j, s16]
            hu16 = plsc.load_gather(hu_v, [si])
            hv16 = plsc.load_gather(hv_v, [di])
            s = jnp.maximum(hu16 + hv16, 0.0) + 0.1
            for e in range(16):
                a = s[e]
                row = g * 16 + e
                for jj in range(D // 16):
                    sl = pl.ds(jj * 16, 16)
                    rows[row, sl] = rows[row, sl] * a
            return 0
        lax.fori_loop(0, C // 16, grp_body, 0)
        pltpu.sync_copy(rows, acc_sh.at[dx.at[j]], add=True)

    NPAIR_LOOP = (EPW // C - 1) // 4

    issue_pair(0, sxA, dxA, ipsA)
    wait_pair(sxA, dxA, ipsA)
    issue_gather(sxA, 0, rows0, gs0)
    issue_pair(1, sxB, dxB, ipsB)

    def pipe_body(k, _):
        issue_gather(sxA, 1, rows1, gs1)
        wait_gather(sxA, 0, rows0, gs0)
        compute_scatter(sxA, dxA, 0, rows0)
        wait_pair(sxB, dxB, ipsB)
        issue_gather(sxB, 0, rows0, gs0)
        wait_gather(sxA, 1, rows1, gs1)
        compute_scatter(sxA, dxA, 1, rows1)
        issue_pair(2 * k + 2, sxA, dxA, ipsA)
        issue_gather(sxB, 1, rows1, gs1)
        wait_gather(sxB, 0, rows0, gs0)
        compute_scatter(sxB, dxB, 0, rows0)
        wait_pair(sxA, dxA, ipsA)
        issue_gather(sxA, 0, rows0, gs0)
        wait_gather(sxB, 1, rows1, gs1)
        compute_scatter(sxB, dxB, 1, rows1)
        issue_pair(2 * k + 3, sxB, dxB, ipsB)
        return 0
    lax.fori_loop(0, NPAIR_LOOP, pipe_body, 0)

    wait_gather(sxA, 0, rows0, gs0)
    compute_scatter(sxA, dxA, 0, rows0)
    wait_pair(sxB, dxB, ipsB)

    plsc.subcore_barrier()

    @pl.when(sid < N // OWN)
    def _():
        pltpu.sync_copy(acc_sh.at[pl.ds(sid * OWN, OWN)],
                        part_out.at[pl.ds(cid * N + sid * OWN, OWN)])


def _tc_final_body(part_ref, nd_ref, w_ref, b_ref, out_ref):
    neigh = (part_ref[0:N, :] + part_ref[N:2 * N, :]) * nd_ref[...]
    out_ref[...] = lax.dot_general(
        neigh, w_ref[...], (((1,), (1,)), ((), ())),
        preferred_element_type=jnp.float32) + b_ref[...]


def _tc_final(part, nd2, W, b2):
    return pl.pallas_call(
        _tc_final_body,
        out_shape=jax.ShapeDtypeStruct((N, D), jnp.float32),
    )(part, nd2, W, b2)


def kernel(hidden_feat, node_feat, edge_index, sample_weights, q_probs, W, b):
    src = edge_index[0].astype(jnp.int32).reshape(NW, EPW)
    dst = edge_index[1].astype(jnp.int32).reshape(NW, EPW)
    dpad = jnp.full((NW, C), N, jnp.int32)
    tail = jnp.full((6, C), N, jnp.int32)
    src2d = jnp.concatenate(
        [jnp.concatenate([src, dpad], axis=1).reshape(NW * NCHUNK, C),
         tail], axis=0)
    dst2d = jnp.concatenate(
        [jnp.concatenate([dst, dpad], axis=1).reshape(NW * NCHUNK, C),
         tail], axis=0)

    hu, hv = _tc_matvec(node_feat, sample_weights)
    deg_part = _sc_degree(dst2d)
    nd, hprime = _tc_scalars(hidden_feat, q_probs.reshape(N, 1), deg_part)
    z16 = jnp.zeros(16, jnp.float32)
    hu_pad = jnp.concatenate([hu.reshape(N), z16])
    hv_pad = jnp.concatenate([hv.reshape(N), z16])
    zeros = jnp.zeros((ZR, D), jnp.float32)
    part = _sc_edges(hprime, src2d, dst2d, hu_pad, hv_pad, zeros)
    return _tc_final(part, nd, W, b.reshape(1, D))
